# Initial kernel scaffold; baseline (speedup 1.0000x reference)
import jax
import jax.numpy as jnp
from jax.experimental import pallas as pl


def kernel(x, edge_index, edge_attr, u, params):
    raise NotImplementedError("write your pallas kernel here")


# speedup vs baseline: 2.8221x; 2.8221x over previous
import functools

import jax
import jax.numpy as jnp
from jax import lax
from jax.experimental import pallas as pl
from jax.experimental.pallas import tpu as pltpu
from jax.experimental.pallas import tpu_sc as plsc

N = 10000
E = 320000
ND = 128
ED = 16
GD = 64
HID = 64

NCORE = 2
NSUB = 16
NW = NCORE * NSUB
EPT = E // NW
CH = 80
NCH = EPT // CH
NPT = N // NSUB

@functools.lru_cache(maxsize=1)
def _mesh():
    return plsc.VectorSubcoreMesh(
        core_axis_name="c", subcore_axis_name="s",
        num_cores=NCORE, num_subcores=NSUB)


def _proj_tc(x, ws, wd):
    def body(x_ref, ws_ref, wd_ref, gs_ref, gd_ref):
        xv = x_ref[...]
        gs_ref[...] = jnp.dot(xv, ws_ref[...], preferred_element_type=jnp.float32)
        gd_ref[...] = jnp.dot(xv, wd_ref[...], preferred_element_type=jnp.float32)
    return pl.pallas_call(
        body,
        out_shape=(jax.ShapeDtypeStruct((N, HID), jnp.float32),
                   jax.ShapeDtypeStruct((N, HID), jnp.float32)),
    )(x, ws, wd)


def _gather_sc(gs, gd, src2d, dst2d):
    @functools.partial(
        pl.kernel,
        out_type=jax.ShapeDtypeStruct((E, HID), jnp.float32),
        mesh=_mesh(),
        scratch_types=[
            pltpu.VMEM((NCH, CH), jnp.int32),
            pltpu.VMEM((NCH, CH), jnp.int32),
            pltpu.VMEM((CH, HID), jnp.float32),
            pltpu.SemaphoreType.DMA,
            pltpu.SemaphoreType.DMA,
        ],
        compiler_params=pltpu.CompilerParams(use_tc_tiling_on_sc=False),
    )
    def k(gs_hbm, gd_hbm, src_hbm, dst_hbm, out_hbm, sidx, didx, buf, s1, s2):
        wid = lax.axis_index("s") * NCORE + lax.axis_index("c")
        pltpu.sync_copy(src_hbm.at[wid], sidx)
        pltpu.sync_copy(dst_hbm.at[wid], didx)
        base = wid * EPT

        def body(i, carry):
            pltpu.async_copy(gs_hbm.at[sidx.at[i]], buf, s1).wait()
            pltpu.async_copy(gd_hbm.at[didx.at[i]], buf, s2, add=True).wait()
            pltpu.sync_copy(buf, out_hbm.at[pl.ds(base + i * CH, CH)])
            return carry

        lax.fori_loop(0, NCH, body, 0)

    return k(gs, gd, src2d, dst2d)


def _edge_tc(gsum, ea, u2, weu, b1e, wee, w2e, b2e):
    BE = 8000
    grid = (E // BE,)

    def body(gsum_ref, ea_ref, u_ref, weu_ref, b1_ref, wee_ref, w2_ref,
             b2_ref, out_ref):
        cvec = jnp.dot(u_ref[...], weu_ref[...],
                       preferred_element_type=jnp.float32) + b1_ref[...]
        h = jnp.maximum(
            gsum_ref[...]
            + jnp.dot(ea_ref[...], wee_ref[...],
                      preferred_element_type=jnp.float32)
            + cvec, 0.0)
        out_ref[...] = jnp.dot(h, w2_ref[...],
                               preferred_element_type=jnp.float32) + b2_ref[...]

    return pl.pallas_call(
        body,
        grid=grid,
        in_specs=[
            pl.BlockSpec((BE, HID), lambda i: (i, 0)),
            pl.BlockSpec((BE, ED), lambda i: (i, 0)),
            pl.BlockSpec((1, GD), lambda i: (0, 0)),
            pl.BlockSpec((GD, HID), lambda i: (0, 0)),
            pl.BlockSpec((1, HID), lambda i: (0, 0)),
            pl.BlockSpec((ED, HID), lambda i: (0, 0)),
            pl.BlockSpec((HID, ED), lambda i: (0, 0)),
            pl.BlockSpec((1, ED), lambda i: (0, 0)),
        ],
        out_specs=pl.BlockSpec((BE, ED), lambda i: (i, 0)),
        out_shape=jax.ShapeDtypeStruct((E, ED), jnp.float32),
    )(gsum, ea, u2, weu, b1e, wee, w2e, b2e)


def _scatter_sc(e_new, dst2d):
    @functools.partial(
        pl.kernel,
        out_type=jax.ShapeDtypeStruct((NCORE, N, ED), jnp.float32),
        mesh=_mesh(),
        scratch_types=[
            pltpu.VMEM((NCH, CH), jnp.int32),
            pltpu.VMEM((CH, ED), jnp.float32),
            pltpu.VMEM((CH, ED), jnp.float32),
            pltpu.VMEM_SHARED((N, ED), jnp.float32),
        ],
        compiler_params=pltpu.CompilerParams(use_tc_tiling_on_sc=False),
    )
    def k(e_hbm, dst_hbm, out_hbm, didx, rowbuf, nbuf, agg_sh):
        cid = lax.axis_index("c")
        sid = lax.axis_index("s")
        wid = sid * NCORE + cid
        nagg = N // CH
        nrounds = (nagg + NSUB - 1) // NSUB

        def zbody(i, carry):
            nbuf[i] = jnp.zeros((ED,), jnp.float32)
            return carry

        lax.fori_loop(0, CH, zbody, 0)

        def zchunk(j, carry):
            c = sid + j * NSUB

            @pl.when(c < nagg)
            def _():
                pltpu.sync_copy(nbuf, agg_sh.at[pl.ds(c * CH, CH)])
            return carry

        lax.fori_loop(0, nrounds, zchunk, 0)
        plsc.subcore_barrier()

        pltpu.sync_copy(dst_hbm.at[wid], didx)

        def body(i, carry):
            pltpu.sync_copy(e_hbm.at[pl.ds(wid * EPT + i * CH, CH)], rowbuf)
            pltpu.sync_copy(rowbuf, agg_sh.at[didx.at[i]], add=True)
            return carry

        lax.fori_loop(0, NCH, body, 0)
        plsc.subcore_barrier()

        def ochunk(j, carry):
            c = sid + j * NSUB

            @pl.when(c < nagg)
            def _():
                pltpu.sync_copy(agg_sh.at[pl.ds(c * CH, CH)], nbuf)
                pltpu.sync_copy(nbuf, out_hbm.at[cid, pl.ds(c * CH, CH)])
            return carry

        lax.fori_loop(0, nrounds, ochunk, 0)

    return k(e_new, dst2d)


def _node_tc(x, aggp, u2, wnx, wna, wnu, b1n, w2n, b2n,
             wgx, wge, wgu, b1g, w2g, b2g, ws_next=None, wd_next=None):
    with_proj = ws_next is not None

    def body(x_ref, aggp_ref, u_ref, wnx_ref, wna_ref, wnu_ref, b1n_ref,
             w2n_ref, b2n_ref, wgx_ref, wge_ref, wgu_ref, b1g_ref, w2g_ref,
             b2g_ref, *rest):
        if with_proj:
            wsn_ref, wdn_ref, xn_ref, un_ref, gs_ref, gd_ref = rest
        else:
            xn_ref, un_ref = rest
        agg = aggp_ref[0] + aggp_ref[1]
        uv = u_ref[...]
        cn = jnp.dot(uv, wnu_ref[...], preferred_element_type=jnp.float32) \
            + b1n_ref[...]
        hn = jnp.maximum(
            jnp.dot(x_ref[...], wnx_ref[...], preferred_element_type=jnp.float32)
            + jnp.dot(agg, wna_ref[...], preferred_element_type=jnp.float32)
            + cn, 0.0)
        xn = jnp.dot(hn, w2n_ref[...], preferred_element_type=jnp.float32) \
            + b2n_ref[...]
        xn_ref[...] = xn
        mean_x = jnp.mean(xn, axis=0, keepdims=True)
        mean_e = jnp.sum(agg, axis=0, keepdims=True) * (1.0 / E)
        hg = jnp.maximum(
            jnp.dot(mean_x, wgx_ref[...], preferred_element_type=jnp.float32)
            + jnp.dot(mean_e, wge_ref[...], preferred_element_type=jnp.float32)
            + jnp.dot(uv, wgu_ref[...], preferred_element_type=jnp.float32)
            + b1g_ref[...], 0.0)
        un_ref[...] = jnp.dot(hg, w2g_ref[...],
                              preferred_element_type=jnp.float32) + b2g_ref[...]
        if with_proj:
            gs_ref[...] = jnp.dot(xn, wsn_ref[...],
                                  preferred_element_type=jnp.float32)
            gd_ref[...] = jnp.dot(xn, wdn_ref[...],
                                  preferred_element_type=jnp.float32)

    outs = [jax.ShapeDtypeStruct((N, ND), jnp.float32),
            jax.ShapeDtypeStruct((1, GD), jnp.float32)]
    args = [x, aggp, u2, wnx, wna, wnu, b1n, w2n, b2n,
            wgx, wge, wgu, b1g, w2g, b2g]
    if with_proj:
        outs += [jax.ShapeDtypeStruct((N, HID), jnp.float32),
                 jax.ShapeDtypeStruct((N, HID), jnp.float32)]
        args += [ws_next, wd_next]
    return pl.pallas_call(body, out_shape=tuple(outs))(*args)


def kernel(x, edge_index, edge_attr, u, params):
    src2d = edge_index[0].reshape(NW, NCH, CH)
    dst2d = edge_index[1].reshape(NW, NCH, CH)
    u2 = u.reshape(1, GD)
    ea = edge_attr
    gs = gd = None
    nlayers = len(params)
    for li, p in enumerate(params):
        we1, be1, we2, be2 = p['edge']
        wn1, bn1, wn2, bn2 = p['node']
        wg1, bg1, wg2, bg2 = p['global']
        ws, wd = we1[:ND], we1[ND:2 * ND]
        wee, weu = we1[2 * ND:2 * ND + ED], we1[2 * ND + ED:]
        wnx, wna, wnu = wn1[:ND], wn1[ND:ND + ED], wn1[ND + ED:]
        wgx, wge, wgu = wg1[:ND], wg1[ND:ND + ED], wg1[ND + ED:]

        if li == 0:
            gs, gd = _proj_tc(x, ws, wd)
        gsum = _gather_sc(gs, gd, src2d, dst2d)
        ea = _edge_tc(gsum, ea, u2, weu, be1.reshape(1, HID), wee, we2,
                      be2.reshape(1, ED))
        aggp = _scatter_sc(ea, dst2d)
        if li + 1 < nlayers:
            wnext = params[li + 1]['edge'][0]
            x, u2, gs, gd = _node_tc(
                x, aggp, u2, wnx, wna, wnu, bn1.reshape(1, HID), wn2,
                bn2.reshape(1, ND), wgx, wge, wgu, bg1.reshape(1, HID), wg2,
                bg2.reshape(1, GD), wnext[:ND], wnext[ND:2 * ND])
        else:
            x, u2 = _node_tc(
                x, aggp, u2, wnx, wna, wnu, bn1.reshape(1, HID), wn2,
                bn2.reshape(1, ND), wgx, wge, wgu, bg1.reshape(1, HID), wg2,
                bg2.reshape(1, GD))
    return (x, ea, u2.reshape(GD))
